# Initial kernel scaffold; baseline (speedup 1.0000x reference)
import jax
import jax.numpy as jnp
from jax.experimental import pallas as pl


def kernel(input_ids, attention_mask, tok_embs, tok_embs_weights):
    raise NotImplementedError("write your pallas kernel here")


# speedup vs baseline: 2.9284x; 2.9284x over previous
import functools

import jax
import jax.numpy as jnp
from jax import lax
from jax.experimental import pallas as pl
from jax.experimental.pallas import tpu as pltpu
from jax.experimental.pallas import tpu_sc as plsc

VOCAB = 30522
VPAD = 30528
DIM = 768
B, L = 16384, 20
LANES = 16

NW = 32
QPW = B // NW
C = 4
CW = C * L
NCHUNK = QPW // C


def _sc_kernel(ids_hbm, am_hbm, wtab_hbm, emb_hbm, out_hbm,
               wtab_v, idx_v, am_v, rows_v, out_v, sem):
    wid = lax.axis_index("s") * 2 + lax.axis_index("c")

    pltpu.sync_copy(wtab_hbm, wtab_v)

    lane = lax.iota(jnp.int32, LANES)
    OFF2 = L - LANES
    mask_hi = lane >= (LANES - OFF2)

    def chunk_body(i, carry):
        base = wid * QPW + i * C
        b20 = base * L
        pltpu.sync_copy(ids_hbm.at[pl.ds(b20, CW)], idx_v)
        pltpu.sync_copy(am_hbm.at[pl.ds(b20, CW)], am_v)
        pltpu.async_copy(emb_hbm.at[idx_v], rows_v, sem).wait()

        for q in range(C):
            ids0 = idx_v[pl.ds(q * L, LANES)]
            ids1 = idx_v[pl.ds(q * L + OFF2, LANES)]
            g0 = plsc.load_gather(wtab_v, [ids0])
            g1 = plsc.load_gather(wtab_v, [ids1])
            m = jnp.maximum(jnp.max(g0), jnp.max(g1))
            e0 = jnp.exp(g0 - m)
            e1 = jnp.exp(g1 - m)
            s = jnp.sum(e0) + jnp.sum(jnp.where(mask_hi, e1, jnp.float32(0.0)))
            inv = jnp.float32(1.0) / lax.broadcast(s, (LANES,))
            am0 = am_v[pl.ds(q * L, LANES)].astype(jnp.float32)
            am1 = am_v[pl.ds(q * L + OFF2, LANES)].astype(jnp.float32)
            w0 = e0 * inv * am0
            w1 = e1 * inv * am1

            wvec = ([lax.broadcast(w0[l], (LANES,)) for l in range(LANES)]
                    + [lax.broadcast(w1[LANES - OFF2 + k], (LANES,))
                       for k in range(OFF2)])

            def jbody(j, _, q=q, wvec=wvec):
                col = j * LANES
                acc = wvec[0] * rows_v[q * L, pl.ds(col, LANES)]
                for l in range(1, L):
                    acc = acc + wvec[l] * rows_v[q * L + l, pl.ds(col, LANES)]
                out_v[q, pl.ds(col, LANES)] = acc
                return 0

            lax.fori_loop(0, DIM // LANES, jbody, 0)

        pltpu.sync_copy(out_v, out_hbm.at[pl.ds(base, C)])
        return carry

    lax.fori_loop(0, NCHUNK, chunk_body, 0)


@jax.jit
def kernel(input_ids, attention_mask, tok_embs, tok_embs_weights):
    ids_flat = input_ids.reshape(-1).astype(jnp.int32)
    am_flat = attention_mask.reshape(-1).astype(jnp.int32)
    wtab = jnp.pad(tok_embs_weights.astype(jnp.float32), (0, VPAD - VOCAB))

    mesh = plsc.VectorSubcoreMesh(core_axis_name="c", subcore_axis_name="s")
    f = pl.kernel(
        _sc_kernel, mesh=mesh,
        compiler_params=pltpu.CompilerParams(needs_layout_passes=False),
        out_type=jax.ShapeDtypeStruct((B, DIM), jnp.float32),
        scratch_types=[
            pltpu.VMEM((VPAD,), jnp.float32),
            pltpu.VMEM((CW,), jnp.int32),
            pltpu.VMEM((CW,), jnp.int32),
            pltpu.VMEM((CW, DIM), jnp.float32),
            pltpu.VMEM((C, DIM), jnp.float32),
            pltpu.SemaphoreType.DMA,
        ],
    )
    return f(ids_flat, am_flat, wtab, tok_embs)


# speedup vs baseline: 4.8492x; 1.6559x over previous
import jax
import jax.numpy as jnp
from jax import lax
from jax.experimental import pallas as pl
from jax.experimental.pallas import tpu as pltpu
from jax.experimental.pallas import tpu_sc as plsc

VOCAB = 30522
VPAD = 30528
DIM = 768
B, L = 16384, 20
LANES = 16
OFF2 = L - LANES

NW = 32
QPW = B // NW
C = 2
CW = C * L
NCHUNK = QPW // C


def _sc_kernel(ids_hbm, am_hbm, wtab_hbm, emb_hbm, out_hbm,
               wtab_v, ids_v, am_v, rows0_v, rows1_v, outb0_v, outb1_v,
               gsem0, gsem1, osem0, osem1):
    wid = lax.axis_index("s") * 2 + lax.axis_index("c")

    pltpu.sync_copy(wtab_hbm, wtab_v)
    pltpu.sync_copy(ids_hbm.at[pl.ds(wid * QPW * L, QPW * L)], ids_v)
    pltpu.sync_copy(am_hbm.at[pl.ds(wid * QPW * L, QPW * L)], am_v)

    lane = lax.iota(jnp.int32, LANES)
    mask_hi = lane >= (LANES - OFF2)

    def fire_gather(i, buf, sem):
        pltpu.async_copy(emb_hbm.at[ids_v.at[pl.ds(i * CW, CW)]], buf, sem)

    def wait_gather(i, buf, sem):
        pltpu.make_async_copy(emb_hbm.at[ids_v.at[pl.ds(i * CW, CW)]],
                              buf, sem).wait()

    def compute_chunk(i, rows_v, out_v):
        for q in range(C):
            ids0 = ids_v[pl.ds(i * CW + q * L, LANES)]
            ids1 = ids_v[pl.ds(i * CW + q * L + OFF2, LANES)]
            g0 = plsc.load_gather(wtab_v, [ids0])
            g1 = plsc.load_gather(wtab_v, [ids1])
            m = jnp.maximum(jnp.max(g0), jnp.max(g1))
            e0 = jnp.exp(g0 - m)
            e1 = jnp.exp(g1 - m)
            s = jnp.sum(e0) + jnp.sum(jnp.where(mask_hi, e1, jnp.float32(0.0)))
            inv = jnp.float32(1.0) / lax.broadcast(s, (LANES,))
            am0 = am_v[pl.ds(i * CW + q * L, LANES)].astype(jnp.float32)
            am1 = am_v[pl.ds(i * CW + q * L + OFF2, LANES)].astype(jnp.float32)
            w0 = e0 * inv * am0
            w1 = e1 * inv * am1

            wvec = ([lax.broadcast(w0[l], (LANES,)) for l in range(LANES)]
                    + [lax.broadcast(w1[LANES - OFF2 + k], (LANES,))
                       for k in range(OFF2)])

            def jbody(j, _, q=q, wvec=wvec):
                col = j * LANES
                acc = wvec[0] * rows_v[q * L, pl.ds(col, LANES)]
                for l in range(1, L):
                    acc = acc + wvec[l] * rows_v[q * L + l, pl.ds(col, LANES)]
                out_v[q, pl.ds(col, LANES)] = acc
                return 0

            lax.fori_loop(0, DIM // LANES, jbody, 0)

    def fire_out(i, out_v, sem):
        pltpu.async_copy(out_v, out_hbm.at[pl.ds(wid * QPW + i * C, C)], sem)

    def wait_out(i, out_v, sem):
        pltpu.make_async_copy(out_v, out_hbm.at[pl.ds(wid * QPW + i * C, C)],
                              sem).wait()

    bufs = ((rows0_v, outb0_v, gsem0, osem0), (rows1_v, outb1_v, gsem1, osem1))

    fire_gather(0, rows0_v, gsem0)

    def loop_body(g, carry):
        for b in range(2):
            i = g * 2 + b
            rows_v, out_v, gsem, osem = bufs[b]
            nrows_v, _, ngsem, _ = bufs[1 - b]
            nxt = i + 1
            if b == 1:
                nxt = jnp.where(nxt < NCHUNK, nxt, 0)
            fire_gather(nxt, nrows_v, ngsem)
            wait_gather(i, rows_v, gsem)
            pl.when(g >= 1)(lambda: wait_out(i, out_v, osem))
            compute_chunk(i, rows_v, out_v)
            fire_out(i, out_v, osem)
        return carry

    lax.fori_loop(0, NCHUNK // 2, loop_body, 0)

    wait_gather(0, rows0_v, gsem0)
    wait_out(NCHUNK - 2, outb0_v, osem0)
    wait_out(NCHUNK - 1, outb1_v, osem1)


@jax.jit
def kernel(input_ids, attention_mask, tok_embs, tok_embs_weights):
    ids_flat = input_ids.reshape(-1).astype(jnp.int32)
    am_flat = attention_mask.reshape(-1).astype(jnp.int32)
    wtab = jnp.pad(tok_embs_weights.astype(jnp.float32), (0, VPAD - VOCAB))

    mesh = plsc.VectorSubcoreMesh(core_axis_name="c", subcore_axis_name="s")
    f = pl.kernel(
        _sc_kernel, mesh=mesh,
        compiler_params=pltpu.CompilerParams(needs_layout_passes=False),
        out_type=jax.ShapeDtypeStruct((B, DIM), jnp.float32),
        scratch_types=[
            pltpu.VMEM((VPAD,), jnp.float32),
            pltpu.VMEM((QPW * L,), jnp.int32),
            pltpu.VMEM((QPW * L,), jnp.int32),
            pltpu.VMEM((CW, DIM), jnp.float32),
            pltpu.VMEM((CW, DIM), jnp.float32),
            pltpu.VMEM((C, DIM), jnp.float32),
            pltpu.VMEM((C, DIM), jnp.float32),
            pltpu.SemaphoreType.DMA,
            pltpu.SemaphoreType.DMA,
            pltpu.SemaphoreType.DMA,
            pltpu.SemaphoreType.DMA,
        ],
    )
    return f(ids_flat, am_flat, wtab, tok_embs)
